# baseline (speedup 1.0000x reference)
import jax
import jax.numpy as jnp
from jax import lax
from jax.experimental import pallas as pl
from jax.experimental.pallas import tpu as pltpu
from jax.experimental.pallas import tpu_sc as plsc

DIM = 64
SEQ = 77
BATCH = 16384

NC = 2
NS = 16
NW = NC * NS
LANES = 16

BT = BATCH // 128
BLOCKS = SEQ * BT
B_PER_W = BLOCKS // NW
NSTEP = B_PER_W // 2


def _sc_body(x_hbm, table_hbm, pos_hbm, out_hbm,
             pos_v, i0, i1, r0, r1, b0, b1,
             gather_sem, idx_sem, osem0, osem1):
    wid = lax.axis_index("s") * NC + lax.axis_index("c")
    g_base = wid * B_PER_W

    pltpu.sync_copy(pos_hbm, pos_v)

    lane = lax.iota(jnp.int32, 16)
    rows = [(lane // 8) + 2 * j for j in range(DIM // LANES)]
    cols = [(lane % 8) * 128 for _ in range(DIM // LANES)]

    def fetch_idx(g, idx_v):
        s = g // BT
        bt = g - s * BT
        return pltpu.async_copy(
            x_hbm.at[s, pl.ds(bt * 128, 128)], idx_v, idx_sem)

    def wait_idx(idx_v):
        pltpu.make_async_copy(x_hbm.at[0, pl.ds(0, 128)], idx_v,
                              idx_sem).wait()

    def start_gather(idx_v, rows_v):
        pltpu.async_copy(table_hbm.at[idx_v], rows_v, gather_sem)

    def wait_gather(rows_v):
        pltpu.make_async_copy(table_hbm.at[pl.ds(0, 128)], rows_v,
                              gather_sem).wait()

    def compute(g, rows_v, blk_v):
        s = g // BT
        p = [pos_v[s, pl.ds(j * LANES, LANES)] for j in range(DIM // LANES)]

        def row_body(r, _):
            for j in range(DIM // LANES):
                v = rows_v[r, pl.ds(j * LANES, LANES)] + p[j]
                plsc.store_scatter(blk_v, [rows[j], cols[j] + r], v)
            return 0
        lax.fori_loop(0, 128, row_body, 0)

    def start_out(g, blk_v, osem):
        s = g // BT
        bt = g - s * BT
        pltpu.async_copy(blk_v, out_hbm.at[s, :, bt], osem)

    def wait_out(blk_v, osem):
        pltpu.make_async_copy(blk_v, out_hbm.at[0, :, 0], osem).wait()

    fetch_idx(g_base, i0).wait()
    start_gather(i0, r0)
    fetch_idx(g_base + 1, i1)

    def step_body(i, _):
        ga = g_base + 2 * i
        gb = ga + 1

        wait_gather(r0)
        wait_idx(i1)
        start_gather(i1, r1)

        @pl.when(i < NSTEP - 1)
        def _():
            fetch_idx(ga + 2, i0)

        @pl.when(i >= 1)
        def _():
            wait_out(b0, osem0)
        compute(ga, r0, b0)
        start_out(ga, b0, osem0)

        wait_gather(r1)

        @pl.when(i < NSTEP - 1)
        def _():
            wait_idx(i0)
            start_gather(i0, r0)
            fetch_idx(gb + 2, i1)

        @pl.when(i >= 1)
        def _():
            wait_out(b1, osem1)
        compute(gb, r1, b1)
        start_out(gb, b1, osem1)
        return 0

    lax.fori_loop(0, NSTEP, step_body, 0)

    wait_out(b0, osem0)
    wait_out(b1, osem1)


@jax.jit
def kernel(x, token_table, position_embedding):
    batch, seq = x.shape
    xT = x.T.astype(jnp.int32)
    pos = position_embedding.reshape(position_embedding.shape[1], DIM)

    run = pl.kernel(
        _sc_body,
        out_type=jax.ShapeDtypeStruct((seq, DIM // 8, batch // 128, 8 * 128),
                                      jnp.float32),
        mesh=plsc.VectorSubcoreMesh(
            core_axis_name="c", subcore_axis_name="s",
            num_cores=NC, num_subcores=NS),
        compiler_params=pltpu.CompilerParams(use_tc_tiling_on_sc=False,
                                             needs_layout_passes=False),
        scratch_types=[
            pltpu.VMEM((seq, DIM), jnp.float32),
            pltpu.VMEM((128,), jnp.int32),
            pltpu.VMEM((128,), jnp.int32),
            pltpu.VMEM((128, DIM), jnp.float32),
            pltpu.VMEM((128, DIM), jnp.float32),
            pltpu.VMEM((DIM // 8, 8 * 128), jnp.float32),
            pltpu.VMEM((DIM // 8, 8 * 128), jnp.float32),
            pltpu.SemaphoreType.DMA,
            pltpu.SemaphoreType.DMA,
            pltpu.SemaphoreType.DMA,
            pltpu.SemaphoreType.DMA,
        ],
    )
    out5 = run(xT, token_table, pos)
    return (out5.reshape(seq, DIM // 8, batch // 128, 8, 128)
                .transpose(2, 4, 0, 1, 3)
                .reshape(batch, seq, DIM))
